# baseline (speedup 1.0000x reference)
import functools

import jax
import jax.numpy as jnp
import numpy as np
from jax import lax
from jax.experimental import pallas as pl
from jax.experimental.pallas import tpu as pltpu
from jax.experimental.pallas import tpu_sc as plsc

N = 10000
E = 320000
D = 128
G = 16

NP = 10240
NDUMP = N
EP = 327680
NC = 2
NS = 16
NW = NC * NS
CH = 128
EPT = EP // NW
NCHUNK = EPT // CH
RPT = NP // NS
QA = 54
QB = 2 * NCHUNK - QA
ROWB = 1280
NROWB = NP // ROWB
SR, SL = 8, 1280
INT_MIN = np.int32(-2147483648)
BIG = 3.0e38


def _sc_rows_body(y_hbm, src_hbm, dst_hbm, keep_hbm, out_hbm,
                  acc, keep_v, sbuf, draw, deff, rows, zrow, sem0, sem1):
    c = lax.axis_index("c")
    s = lax.axis_index("s")
    wid = s * NC + c
    for i in range(8):
        for j in range(D // 16):
            zrow[i, pl.ds(j * 16, 16)] = jnp.zeros((16,), jnp.float32)
    rbase = s * RPT

    @pl.loop(0, RPT // 8)
    def _zero(i):
        pltpu.sync_copy(zrow, acc.at[pl.ds(rbase + i * 8, 8)])

    pltpu.sync_copy(keep_hbm, keep_v)
    plsc.subcore_barrier()
    nch = jnp.where(c == 0, QA, QB)
    ebase = jnp.where(c == 0, s * (QA * CH), NS * QA * CH + s * (QB * CH))
    sems = (sem0, sem1)

    def load_and_prep(chunk, b):
        base = ebase + chunk * CH
        pltpu.sync_copy(src_hbm.at[pl.ds(base, CH)], sbuf.at[b])
        pltpu.sync_copy(dst_hbm.at[pl.ds(base, CH)], draw)
        for j in range(CH // 16):
            si = sbuf[b, pl.ds(j * 16, 16)]
            di = draw[pl.ds(j * 16, 16)]
            ks = plsc.load_gather(keep_v, [si])
            kd = plsc.load_gather(keep_v, [di])
            ok = jnp.logical_and(ks > 0, kd > 0)
            deff[b, pl.ds(j * 16, 16)] = jnp.where(ok, di, NDUMP)

    load_and_prep(0, 0)
    pltpu.async_copy(y_hbm.at[sbuf.at[0]], rows.at[0], sem0)

    @pl.loop(0, nch // 2)
    def _pair(i):
        for b in range(2):
            cur = 2 * i + b
            nb = 1 - b

            @pl.when(cur + 1 < nch)
            def _():
                load_and_prep(cur + 1, nb)
                pltpu.async_copy(y_hbm.at[sbuf.at[nb]], rows.at[nb], sems[nb])

            pltpu.make_async_copy(y_hbm.at[sbuf.at[b]], rows.at[b],
                                  sems[b]).wait()
            pltpu.sync_copy(rows.at[b], acc.at[deff.at[b]], add=True)

    plsc.subcore_barrier()
    pltpu.sync_copy(acc.at[pl.ds(rbase, RPT)], out_hbm.at[c, pl.ds(rbase, RPT)])


def _sc_scalar_body(vals_hbm, keep_hbm, src_hbm, dst_hbm, out_hbm,
                    acc1, vals_v, keep_v, sbuf, dbuf, vbuf, dbuf2, zb):
    c = lax.axis_index("c")
    s = lax.axis_index("s")
    wid = s * NC + c
    for j in range(8):
        zb[pl.ds(j * 16, 16)] = jnp.zeros((16,), jnp.float32)
    zbase = s * (NP // NS)

    @pl.loop(0, NP // NS // CH)
    def _zero(i):
        pltpu.sync_copy(zb, acc1.at[pl.ds(zbase + i * CH, CH)])

    pltpu.sync_copy(vals_hbm, vals_v)
    pltpu.sync_copy(keep_hbm, keep_v)
    plsc.subcore_barrier()
    ebase = wid * EPT

    @pl.loop(0, NCHUNK)
    def _chunk(g):
        base = ebase + g * CH
        pltpu.sync_copy(src_hbm.at[pl.ds(base, CH)], sbuf)
        pltpu.sync_copy(dst_hbm.at[pl.ds(base, CH)], dbuf)
        for j in range(CH // 16):
            si = sbuf[pl.ds(j * 16, 16)]
            di = dbuf[pl.ds(j * 16, 16)]
            ks = plsc.load_gather(keep_v, [si])
            kd = plsc.load_gather(keep_v, [di])
            pv = plsc.load_gather(vals_v, [si])
            val = jnp.where(jnp.logical_and(ks > 0, kd > 0), pv, 0.0)
            vbuf[0, pl.ds(j * 16, 16)] = val
            dbuf2[0, pl.ds(j * 16, 16)] = di
        pltpu.sync_copy(vbuf.at[0], acc1.at[dbuf2.at[0]], add=True)

    plsc.subcore_barrier()
    pltpu.sync_copy(acc1.at[pl.ds(zbase, NP // NS)],
                    out_hbm.at[c, pl.ds(zbase, NP // NS)])


@functools.cache
def _sc_kernels():
    mesh = plsc.VectorSubcoreMesh(core_axis_name="c", subcore_axis_name="s",
                                  num_cores=NC, num_subcores=NS)
    params = pltpu.CompilerParams(needs_layout_passes=False)
    sc_rows = pl.kernel(
        _sc_rows_body,
        out_type=jax.ShapeDtypeStruct((NC, NP, D), jnp.float32),
        mesh=mesh,
        compiler_params=params,
        scratch_types=[
            pltpu.VMEM_SHARED((NP, D), jnp.float32),
            pltpu.VMEM((NP,), jnp.int32),
            pltpu.VMEM((2, CH), jnp.int32),
            pltpu.VMEM((CH,), jnp.int32),
            pltpu.VMEM((2, CH), jnp.int32),
            pltpu.VMEM((2, CH, D), jnp.float32),
            pltpu.VMEM((8, D), jnp.float32),
            pltpu.SemaphoreType.DMA,
            pltpu.SemaphoreType.DMA,
        ],
    )
    sc_scalar = pl.kernel(
        _sc_scalar_body,
        out_type=jax.ShapeDtypeStruct((NC, NP), jnp.float32),
        mesh=mesh,
        compiler_params=params,
        scratch_types=[
            pltpu.VMEM_SHARED((NP,), jnp.float32),
            pltpu.VMEM((NP,), jnp.float32),
            pltpu.VMEM((NP,), jnp.int32),
            pltpu.VMEM((CH,), jnp.int32),
            pltpu.VMEM((CH,), jnp.int32),
            pltpu.VMEM((1, CH), jnp.float32),
            pltpu.VMEM((1, CH), jnp.int32),
            pltpu.VMEM((CH,), jnp.float32),
        ],
    )
    return sc_rows, sc_scalar


def _mmul(a, b):
    return jnp.dot(a, b, preferred_element_type=jnp.float32)


def _tca_body(x_ref, w_ref, degt_ref, nm_ref, y_ref, xw_ref, dis_ref):
    xw = _mmul(x_ref[...], w_ref[...])
    deg = jnp.sum(degt_ref[...], axis=1, keepdims=True) + nm_ref[...]
    dis = jnp.where(deg > 0, 1.0 / jnp.sqrt(jnp.maximum(deg, 1e-12)), 0.0)
    xw_ref[...] = xw
    y_ref[...] = xw * dis
    dis_ref[...] = dis


_tca = pl.pallas_call(
    _tca_body,
    grid=(NROWB,),
    in_specs=[
        pl.BlockSpec((ROWB, D), lambda i: (i, 0)),
        pl.BlockSpec((D, D), lambda i: (0, 0)),
        pl.BlockSpec((ROWB, NC), lambda i: (i, 0)),
        pl.BlockSpec((ROWB, 1), lambda i: (i, 0)),
    ],
    out_specs=[
        pl.BlockSpec((ROWB, D), lambda i: (i, 0)),
        pl.BlockSpec((ROWB, D), lambda i: (i, 0)),
        pl.BlockSpec((ROWB, 1), lambda i: (i, 0)),
    ],
    out_shape=[
        jax.ShapeDtypeStruct((NP, D), jnp.float32),
        jax.ShapeDtypeStruct((NP, D), jnp.float32),
        jax.ShapeDtypeStruct((NP, 1), jnp.float32),
    ],
)


def _tcb_body(accp_ref, xw_ref, dis_ref, nm_ref, b_ref, xnew_ref):
    acc = accp_ref[0] + accp_ref[1]
    dis = dis_ref[...]
    nm = nm_ref[...]
    gout = acc * dis + (dis * dis * nm) * xw_ref[...] + b_ref[...]
    xnew_ref[...] = jnp.maximum(gout * nm, 0.0)


_tcb = pl.pallas_call(
    _tcb_body,
    grid=(NROWB,),
    in_specs=[
        pl.BlockSpec((NC, ROWB, D), lambda i: (0, i, 0)),
        pl.BlockSpec((ROWB, D), lambda i: (i, 0)),
        pl.BlockSpec((ROWB, 1), lambda i: (i, 0)),
        pl.BlockSpec((ROWB, 1), lambda i: (i, 0)),
        pl.BlockSpec((1, D), lambda i: (0, 0)),
    ],
    out_specs=pl.BlockSpec((ROWB, D), lambda i: (i, 0)),
    out_shape=jax.ShapeDtypeStruct((NP, D), jnp.float32),
)


def _tcb2_body(aggp_ref, xnew_ref, wrel_ref, wroot_ref, brel_ref, s4_ref):
    agg = aggp_ref[0] + aggp_ref[1]
    s4_ref[...] = (_mmul(agg, wrel_ref[...]) + brel_ref[...]) \
        + _mmul(xnew_ref[...], wroot_ref[...])


_tcb2 = pl.pallas_call(
    _tcb2_body,
    grid=(NROWB,),
    in_specs=[
        pl.BlockSpec((NC, ROWB, D), lambda i: (0, i, 0)),
        pl.BlockSpec((ROWB, D), lambda i: (i, 0)),
        pl.BlockSpec((D, 8), lambda i: (0, 0)),
        pl.BlockSpec((D, 8), lambda i: (0, 0)),
        pl.BlockSpec((1, 1), lambda i: (0, 0)),
    ],
    out_specs=pl.BlockSpec((ROWB, 8), lambda i: (i, 0)),
    out_shape=jax.ShapeDtypeStruct((NP, 8), jnp.float32),
)


def _tcc1_body(s_ref, batch_ref, nm_ref, keep_ref, scale_ref):
    s = s_ref[...]
    nm = nm_ref[...]
    alive = nm > 0.0
    ib = lax.bitcast_convert_type(s, jnp.int32)
    key = jnp.where(ib >= 0, ib, ib ^ np.int32(0x7FFFFFFF))
    key = jnp.where(alive, key, INT_MIN)
    r_i = lax.broadcasted_iota(jnp.int32, (SR, SL), 0)
    c_i = lax.broadcasted_iota(jnp.int32, (SR, SL), 1)
    idxv = r_i * SL + c_i
    gi = lax.broadcasted_iota(jnp.int32, (G, SR, SL), 0)
    g3 = gi == batch_ref[...][None, :, :]
    af = jnp.logical_and(g3, alive[None])
    cnt = jnp.sum(af.astype(jnp.int32), axis=(1, 2))
    k = ((cnt + 1) // 2).reshape(G, 1, 1)
    lo = jnp.full((G, 1, 1), INT_MIN, jnp.int32)
    for b in range(31, -1, -1):
        cand = lo + (INT_MIN if b == 31 else np.int32(1 << b))
        ind = jnp.logical_and(af, key[None] >= cand)
        c2 = jnp.sum(ind.astype(jnp.int32), axis=(1, 2)).reshape(G, 1, 1)
        lo = jnp.where(c2 >= k, cand, lo)
    t = lo
    gtc = jnp.sum(jnp.logical_and(af, key[None] > t).astype(jnp.int32),
                  axis=(1, 2)).reshape(G, 1, 1)
    r = k - gtc
    eqi = jnp.logical_and(af, key[None] == t).astype(jnp.int32)
    u = jnp.zeros((G, 1, 1), jnp.int32)
    for b in range(14, -1, -1):
        cand = u + np.int32(1 << b)
        sc = jnp.sum(jnp.where(idxv[None] < cand, eqi, 0),
                     axis=(1, 2)).reshape(G, 1, 1)
        u = jnp.where(sc <= r, cand, u)
    tv = jnp.sum(jnp.where(g3, t, 0), axis=0)
    uv = jnp.sum(jnp.where(g3, u, 0), axis=0)
    keep = jnp.logical_and(
        alive,
        jnp.logical_or(key > tv,
                       jnp.logical_and(key == tv, idxv < uv)))
    keep_ref[...] = keep.astype(jnp.float32)
    scale_ref[...] = jnp.tanh(s)


_tcc1 = pl.pallas_call(
    _tcc1_body,
    grid=(1,),
    in_specs=[
        pl.BlockSpec((SR, SL), lambda i: (0, 0)),
        pl.BlockSpec((SR, SL), lambda i: (0, 0)),
        pl.BlockSpec((SR, SL), lambda i: (0, 0)),
    ],
    out_specs=[
        pl.BlockSpec((SR, SL), lambda i: (0, 0)),
        pl.BlockSpec((SR, SL), lambda i: (0, 0)),
    ],
    out_shape=[
        jax.ShapeDtypeStruct((SR, SL), jnp.float32),
        jax.ShapeDtypeStruct((SR, SL), jnp.float32),
    ],
)


def _tcc2_body(x_ref, sc_ref, kp_ref, br_ref, bl_ref, xm_ref, h_ref):
    i = pl.program_id(0)
    xm = (x_ref[...] * sc_ref[...]) * kp_ref[...]
    xm_ref[...] = xm

    @pl.when(i == 0)
    def _():
        h_ref[...] = jnp.concatenate(
            [jnp.full((G, D), -BIG, jnp.float32),
             jnp.zeros((G, D), jnp.float32)], axis=1)

    gm = (lax.broadcasted_iota(jnp.int32, (G, ROWB), 0)
          == bl_ref[...]).astype(jnp.float32)
    smp = jnp.dot(gm, xm, preferred_element_type=jnp.float32,
                  precision=lax.Precision.HIGHEST)
    h_ref[:, D:] += smp
    br = br_ref[...]
    kp = kp_ref[...]
    for g in range(G):
        m = jnp.logical_and(br == g, kp > 0.0)
        red = jnp.max(jnp.where(m, xm, -BIG), axis=0, keepdims=True)
        h_ref[g:g + 1, :D] = jnp.maximum(h_ref[g:g + 1, :D], red)

    @pl.when(i == NROWB - 1)
    def _():
        mx = h_ref[:, :D]
        h_ref[:, :D] = jnp.where(mx == -BIG, 0.0, mx)


_tcc2 = pl.pallas_call(
    _tcc2_body,
    grid=(NROWB,),
    in_specs=[
        pl.BlockSpec((ROWB, D), lambda i: (i, 0)),
        pl.BlockSpec((ROWB, 1), lambda i: (i, 0)),
        pl.BlockSpec((ROWB, 1), lambda i: (i, 0)),
        pl.BlockSpec((ROWB, 1), lambda i: (i, 0)),
        pl.BlockSpec((1, ROWB), lambda i: (0, i)),
    ],
    out_specs=[
        pl.BlockSpec((ROWB, D), lambda i: (i, 0)),
        pl.BlockSpec((G, 2 * D), lambda i: (0, 0)),
    ],
    out_shape=[
        jax.ShapeDtypeStruct((NP, D), jnp.float32),
        jax.ShapeDtypeStruct((G, 2 * D), jnp.float32),
    ],
)


def _tcsum_body(a_ref, b_ref, c_ref, o_ref):
    o_ref[...] = a_ref[...] + b_ref[...] + c_ref[...]


_tcsum = pl.pallas_call(
    _tcsum_body,
    grid=(1,),
    in_specs=[pl.BlockSpec((G, 2 * D), lambda i: (0, 0))] * 3,
    out_specs=pl.BlockSpec((G, 2 * D), lambda i: (0, 0)),
    out_shape=jax.ShapeDtypeStruct((G, 2 * D), jnp.float32),
)


def kernel(x, edge_index, batch, hls_attr, W0, b0, W1, b1, W2, b2,
           Wrel0, brel0, Wroot0, Wrel1, brel1, Wroot1, Wrel2, brel2, Wroot2):
    f32 = jnp.float32
    src = jnp.concatenate([edge_index[0], jnp.zeros((EP - E,), jnp.int32)])
    dst = jnp.concatenate([edge_index[1], jnp.full((EP - E,), N, jnp.int32)])
    xp = jnp.pad(x, ((0, NP - N), (0, 0)))
    batch_p = jnp.concatenate([batch, jnp.full((NP - N,), G, jnp.int32)])
    batch2 = batch_p.reshape(SR, SL)
    batch_row = batch_p.reshape(NP, 1)
    batch_lane = batch_p.reshape(1, NP)
    keep_i = (jnp.arange(NP, dtype=jnp.int32) < N).astype(jnp.int32)
    nmask2 = keep_i.astype(f32).reshape(SR, SL)
    nm_row = keep_i.astype(f32).reshape(NP, 1)

    params = [(W0, b0, Wrel0, brel0, Wroot0),
              (W1, b1, Wrel1, brel1, Wroot1),
              (W2, b2, Wrel2, brel2, Wroot2)]

    _sc_rows, _sc_scalar = _sc_kernels()
    degparts = _sc_scalar(jnp.ones((NP,), f32), keep_i, src, dst)
    x_cur = xp
    hs = []
    for l in range(3):
        W, b, Wrel, brel, Wroot = params[l]
        degt = degparts.T
        y, xw, dis = _tca(x_cur, W, degt, nm_row)
        accparts = _sc_rows(y, src, dst, keep_i)
        xnew = _tcb(accparts, xw, dis, nm_row, b.reshape(1, D))
        aggparts = _sc_rows(xnew, src, dst, keep_i)
        wrel8 = jnp.concatenate([Wrel, jnp.zeros((D, 7), f32)], axis=1)
        wroot8 = jnp.concatenate([Wroot, jnp.zeros((D, 7), f32)], axis=1)
        s4 = _tcb2(aggparts, xnew, wrel8, wroot8, brel.reshape(1, 1))
        s2 = s4[:, 0].reshape(SR, SL)
        keep2, tanh2 = _tcc1(s2, batch2, nmask2)
        keep_new_i = keep2.reshape(NP).astype(jnp.int32)
        xmod, h = _tcc2(xnew, tanh2.reshape(NP, 1), keep2.reshape(NP, 1),
                        batch_row, batch_lane)
        hs.append(h)
        if l < 2:
            degparts = _sc_scalar(keep2.reshape(NP), keep_new_i, src, dst)
        x_cur = xmod
        keep_i = keep_new_i
        nmask2 = keep2
        nm_row = keep2.reshape(NP, 1)

    hsum = _tcsum(hs[0], hs[1], hs[2])
    out = jnp.concatenate([hsum, hls_attr], axis=1)
    return (out, hs[0], hs[1], hs[2])


# speedup vs baseline: 1.1391x; 1.1391x over previous
import functools

import jax
import jax.numpy as jnp
import numpy as np
from jax import lax
from jax.experimental import pallas as pl
from jax.experimental.pallas import tpu as pltpu
from jax.experimental.pallas import tpu_sc as plsc

N = 10000
E = 320000
D = 128
G = 16

NP = 10240
NDUMP = N
EP = 327680
NC = 2
NS = 16
NW = NC * NS
CH = 128
EPT = EP // NW
NCHUNK = EPT // CH
RPT = NP // NS
QA = 106
QB = 2 * NCHUNK - QA
ROWB = 1280
NROWB = NP // ROWB
SR, SL = 8, 1280
INT_MIN = np.int32(-2147483648)
BIG = 3.0e38


def _sc_rows_body(y_hbm, src_hbm, dst_hbm, keep_hbm, out_hbm,
                  acc, keep_v, sbuf, draw, deff, rows, zrow, sem0, sem1):
    c = lax.axis_index("c")
    s = lax.axis_index("s")
    wid = s * NC + c
    for i in range(8):
        for j in range(D // 16):
            zrow[i, pl.ds(j * 16, 16)] = jnp.zeros((16,), jnp.float32)
    rbase = s * RPT

    @pl.loop(0, RPT // 8)
    def _zero(i):
        pltpu.sync_copy(zrow, acc.at[pl.ds(rbase + i * 8, 8)])

    pltpu.sync_copy(keep_hbm, keep_v)
    plsc.subcore_barrier()
    nch = jnp.where(c == 0, QA, QB)
    ebase = jnp.where(c == 0, s * (QA * CH), NS * QA * CH + s * (QB * CH))
    sems = (sem0, sem1)

    def load_and_prep(chunk, b):
        base = ebase + chunk * CH
        pltpu.sync_copy(src_hbm.at[pl.ds(base, CH)], sbuf.at[b])
        pltpu.sync_copy(dst_hbm.at[pl.ds(base, CH)], draw)
        for j in range(CH // 16):
            si = sbuf[b, pl.ds(j * 16, 16)]
            di = draw[pl.ds(j * 16, 16)]
            ks = plsc.load_gather(keep_v, [si])
            kd = plsc.load_gather(keep_v, [di])
            ok = jnp.logical_and(ks > 0, kd > 0)
            deff[b, pl.ds(j * 16, 16)] = jnp.where(ok, di, NDUMP)

    load_and_prep(0, 0)
    pltpu.async_copy(y_hbm.at[sbuf.at[0]], rows.at[0], sem0)

    @pl.loop(0, nch // 2)
    def _pair(i):
        for b in range(2):
            cur = 2 * i + b
            nb = 1 - b

            @pl.when(cur + 1 < nch)
            def _():
                load_and_prep(cur + 1, nb)
                pltpu.async_copy(y_hbm.at[sbuf.at[nb]], rows.at[nb], sems[nb])

            pltpu.make_async_copy(y_hbm.at[sbuf.at[b]], rows.at[b],
                                  sems[b]).wait()
            pltpu.sync_copy(rows.at[b], acc.at[deff.at[b]], add=True)

    plsc.subcore_barrier()
    pltpu.sync_copy(acc.at[pl.ds(rbase, RPT)], out_hbm.at[c, pl.ds(rbase, RPT)])


def _sc_scalar_body(vals_hbm, keep_hbm, src_hbm, dst_hbm, out_hbm,
                    acc1, vals_v, keep_v, sbuf, dbuf, vbuf, dbuf2, zb):
    c = lax.axis_index("c")
    s = lax.axis_index("s")
    wid = s * NC + c
    for j in range(8):
        zb[pl.ds(j * 16, 16)] = jnp.zeros((16,), jnp.float32)
    zbase = s * (NP // NS)

    @pl.loop(0, NP // NS // CH)
    def _zero(i):
        pltpu.sync_copy(zb, acc1.at[pl.ds(zbase + i * CH, CH)])

    pltpu.sync_copy(vals_hbm, vals_v)
    pltpu.sync_copy(keep_hbm, keep_v)
    plsc.subcore_barrier()
    ebase = wid * EPT

    @pl.loop(0, NCHUNK)
    def _chunk(g):
        base = ebase + g * CH
        pltpu.sync_copy(src_hbm.at[pl.ds(base, CH)], sbuf)
        pltpu.sync_copy(dst_hbm.at[pl.ds(base, CH)], dbuf)
        for j in range(CH // 16):
            si = sbuf[pl.ds(j * 16, 16)]
            di = dbuf[pl.ds(j * 16, 16)]
            ks = plsc.load_gather(keep_v, [si])
            kd = plsc.load_gather(keep_v, [di])
            pv = plsc.load_gather(vals_v, [si])
            val = jnp.where(jnp.logical_and(ks > 0, kd > 0), pv, 0.0)
            vbuf[0, pl.ds(j * 16, 16)] = val
            dbuf2[0, pl.ds(j * 16, 16)] = di
        pltpu.sync_copy(vbuf.at[0], acc1.at[dbuf2.at[0]], add=True)

    plsc.subcore_barrier()
    pltpu.sync_copy(acc1.at[pl.ds(zbase, NP // NS)],
                    out_hbm.at[c, pl.ds(zbase, NP // NS)])


@functools.cache
def _sc_kernels():
    mesh = plsc.VectorSubcoreMesh(core_axis_name="c", subcore_axis_name="s",
                                  num_cores=NC, num_subcores=NS)
    params = pltpu.CompilerParams(needs_layout_passes=False)
    sc_rows = pl.kernel(
        _sc_rows_body,
        out_type=jax.ShapeDtypeStruct((NC, NP, D), jnp.float32),
        mesh=mesh,
        compiler_params=params,
        scratch_types=[
            pltpu.VMEM_SHARED((NP, D), jnp.float32),
            pltpu.VMEM((NP,), jnp.int32),
            pltpu.VMEM((2, CH), jnp.int32),
            pltpu.VMEM((CH,), jnp.int32),
            pltpu.VMEM((2, CH), jnp.int32),
            pltpu.VMEM((2, CH, D), jnp.float32),
            pltpu.VMEM((8, D), jnp.float32),
            pltpu.SemaphoreType.DMA,
            pltpu.SemaphoreType.DMA,
        ],
    )
    sc_scalar = pl.kernel(
        _sc_scalar_body,
        out_type=jax.ShapeDtypeStruct((NC, NP), jnp.float32),
        mesh=mesh,
        compiler_params=params,
        scratch_types=[
            pltpu.VMEM_SHARED((NP,), jnp.float32),
            pltpu.VMEM((NP,), jnp.float32),
            pltpu.VMEM((NP,), jnp.int32),
            pltpu.VMEM((CH,), jnp.int32),
            pltpu.VMEM((CH,), jnp.int32),
            pltpu.VMEM((1, CH), jnp.float32),
            pltpu.VMEM((1, CH), jnp.int32),
            pltpu.VMEM((CH,), jnp.float32),
        ],
    )
    return sc_rows, sc_scalar


def _mmul(a, b):
    return jnp.dot(a, b, preferred_element_type=jnp.float32)


def _tca_body(x_ref, w_ref, degt_ref, nm_ref, y_ref, xw_ref, dis_ref):
    xw = _mmul(x_ref[...], w_ref[...])
    deg = jnp.sum(degt_ref[...], axis=1, keepdims=True) + nm_ref[...]
    dis = jnp.where(deg > 0, 1.0 / jnp.sqrt(jnp.maximum(deg, 1e-12)), 0.0)
    xw_ref[...] = xw
    y_ref[...] = xw * dis
    dis_ref[...] = dis


_tca = pl.pallas_call(
    _tca_body,
    grid=(NROWB,),
    in_specs=[
        pl.BlockSpec((ROWB, D), lambda i: (i, 0)),
        pl.BlockSpec((D, D), lambda i: (0, 0)),
        pl.BlockSpec((ROWB, NC), lambda i: (i, 0)),
        pl.BlockSpec((ROWB, 1), lambda i: (i, 0)),
    ],
    out_specs=[
        pl.BlockSpec((ROWB, D), lambda i: (i, 0)),
        pl.BlockSpec((ROWB, D), lambda i: (i, 0)),
        pl.BlockSpec((ROWB, 1), lambda i: (i, 0)),
    ],
    out_shape=[
        jax.ShapeDtypeStruct((NP, D), jnp.float32),
        jax.ShapeDtypeStruct((NP, D), jnp.float32),
        jax.ShapeDtypeStruct((NP, 1), jnp.float32),
    ],
)


def _tcb_body(accp_ref, xw_ref, dis_ref, nm_ref, b_ref, xnew_ref):
    acc = accp_ref[0] + accp_ref[1]
    dis = dis_ref[...]
    nm = nm_ref[...]
    gout = acc * dis + (dis * dis * nm) * xw_ref[...] + b_ref[...]
    xnew_ref[...] = jnp.maximum(gout * nm, 0.0)


_tcb = pl.pallas_call(
    _tcb_body,
    grid=(NROWB,),
    in_specs=[
        pl.BlockSpec((NC, ROWB, D), lambda i: (0, i, 0)),
        pl.BlockSpec((ROWB, D), lambda i: (i, 0)),
        pl.BlockSpec((ROWB, 1), lambda i: (i, 0)),
        pl.BlockSpec((ROWB, 1), lambda i: (i, 0)),
        pl.BlockSpec((1, D), lambda i: (0, 0)),
    ],
    out_specs=pl.BlockSpec((ROWB, D), lambda i: (i, 0)),
    out_shape=jax.ShapeDtypeStruct((NP, D), jnp.float32),
)


def _tcb2_body(aggp_ref, xnew_ref, wrel_ref, wroot_ref, brel_ref, s4_ref):
    agg = aggp_ref[0] + aggp_ref[1]
    s4_ref[...] = (_mmul(agg, wrel_ref[...]) + brel_ref[...]) \
        + _mmul(xnew_ref[...], wroot_ref[...])


_tcb2 = pl.pallas_call(
    _tcb2_body,
    grid=(NROWB,),
    in_specs=[
        pl.BlockSpec((NC, ROWB, D), lambda i: (0, i, 0)),
        pl.BlockSpec((ROWB, D), lambda i: (i, 0)),
        pl.BlockSpec((D, 8), lambda i: (0, 0)),
        pl.BlockSpec((D, 8), lambda i: (0, 0)),
        pl.BlockSpec((1, 1), lambda i: (0, 0)),
    ],
    out_specs=pl.BlockSpec((ROWB, 8), lambda i: (i, 0)),
    out_shape=jax.ShapeDtypeStruct((NP, 8), jnp.float32),
)


def _tcc1_body(s_ref, batch_ref, nm_ref, keep_ref, scale_ref):
    s = s_ref[...]
    nm = nm_ref[...]
    alive = nm > 0.0
    ib = lax.bitcast_convert_type(s, jnp.int32)
    key = jnp.where(ib >= 0, ib, ib ^ np.int32(0x7FFFFFFF))
    key = jnp.where(alive, key, INT_MIN)
    r_i = lax.broadcasted_iota(jnp.int32, (SR, SL), 0)
    c_i = lax.broadcasted_iota(jnp.int32, (SR, SL), 1)
    idxv = r_i * SL + c_i
    gi = lax.broadcasted_iota(jnp.int32, (G, SR, SL), 0)
    g3 = gi == batch_ref[...][None, :, :]
    af = jnp.logical_and(g3, alive[None])
    cnt = jnp.sum(af.astype(jnp.int32), axis=(1, 2))
    k = ((cnt + 1) // 2).reshape(G, 1, 1)
    lo = jnp.full((G, 1, 1), INT_MIN, jnp.int32)
    for b in range(31, -1, -1):
        cand = lo + (INT_MIN if b == 31 else np.int32(1 << b))
        ind = jnp.logical_and(af, key[None] >= cand)
        c2 = jnp.sum(ind.astype(jnp.int32), axis=(1, 2)).reshape(G, 1, 1)
        lo = jnp.where(c2 >= k, cand, lo)
    t = lo
    gtc = jnp.sum(jnp.logical_and(af, key[None] > t).astype(jnp.int32),
                  axis=(1, 2)).reshape(G, 1, 1)
    r = k - gtc
    eqi = jnp.logical_and(af, key[None] == t).astype(jnp.int32)
    u = jnp.zeros((G, 1, 1), jnp.int32)
    for b in range(14, -1, -1):
        cand = u + np.int32(1 << b)
        sc = jnp.sum(jnp.where(idxv[None] < cand, eqi, 0),
                     axis=(1, 2)).reshape(G, 1, 1)
        u = jnp.where(sc <= r, cand, u)
    tv = jnp.sum(jnp.where(g3, t, 0), axis=0)
    uv = jnp.sum(jnp.where(g3, u, 0), axis=0)
    keep = jnp.logical_and(
        alive,
        jnp.logical_or(key > tv,
                       jnp.logical_and(key == tv, idxv < uv)))
    keep_ref[...] = keep.astype(jnp.float32)
    scale_ref[...] = jnp.tanh(s)


_tcc1 = pl.pallas_call(
    _tcc1_body,
    grid=(1,),
    in_specs=[
        pl.BlockSpec((SR, SL), lambda i: (0, 0)),
        pl.BlockSpec((SR, SL), lambda i: (0, 0)),
        pl.BlockSpec((SR, SL), lambda i: (0, 0)),
    ],
    out_specs=[
        pl.BlockSpec((SR, SL), lambda i: (0, 0)),
        pl.BlockSpec((SR, SL), lambda i: (0, 0)),
    ],
    out_shape=[
        jax.ShapeDtypeStruct((SR, SL), jnp.float32),
        jax.ShapeDtypeStruct((SR, SL), jnp.float32),
    ],
)


def _tcc2_body(x_ref, sc_ref, kp_ref, br_ref, bl_ref, xm_ref, h_ref):
    i = pl.program_id(0)
    xm = (x_ref[...] * sc_ref[...]) * kp_ref[...]
    xm_ref[...] = xm

    @pl.when(i == 0)
    def _():
        h_ref[...] = jnp.concatenate(
            [jnp.full((G, D), -BIG, jnp.float32),
             jnp.zeros((G, D), jnp.float32)], axis=1)

    gm = (lax.broadcasted_iota(jnp.int32, (G, ROWB), 0)
          == bl_ref[...]).astype(jnp.float32)
    smp = jnp.dot(gm, xm, preferred_element_type=jnp.float32,
                  precision=lax.Precision.HIGHEST)
    h_ref[:, D:] += smp
    br = br_ref[...]
    kp = kp_ref[...]
    for g in range(G):
        m = jnp.logical_and(br == g, kp > 0.0)
        red = jnp.max(jnp.where(m, xm, -BIG), axis=0, keepdims=True)
        h_ref[g:g + 1, :D] = jnp.maximum(h_ref[g:g + 1, :D], red)

    @pl.when(i == NROWB - 1)
    def _():
        mx = h_ref[:, :D]
        h_ref[:, :D] = jnp.where(mx == -BIG, 0.0, mx)


_tcc2 = pl.pallas_call(
    _tcc2_body,
    grid=(NROWB,),
    in_specs=[
        pl.BlockSpec((ROWB, D), lambda i: (i, 0)),
        pl.BlockSpec((ROWB, 1), lambda i: (i, 0)),
        pl.BlockSpec((ROWB, 1), lambda i: (i, 0)),
        pl.BlockSpec((ROWB, 1), lambda i: (i, 0)),
        pl.BlockSpec((1, ROWB), lambda i: (0, i)),
    ],
    out_specs=[
        pl.BlockSpec((ROWB, D), lambda i: (i, 0)),
        pl.BlockSpec((G, 2 * D), lambda i: (0, 0)),
    ],
    out_shape=[
        jax.ShapeDtypeStruct((NP, D), jnp.float32),
        jax.ShapeDtypeStruct((G, 2 * D), jnp.float32),
    ],
)


def _tcsum_body(a_ref, b_ref, c_ref, o_ref):
    o_ref[...] = a_ref[...] + b_ref[...] + c_ref[...]


_tcsum = pl.pallas_call(
    _tcsum_body,
    grid=(1,),
    in_specs=[pl.BlockSpec((G, 2 * D), lambda i: (0, 0))] * 3,
    out_specs=pl.BlockSpec((G, 2 * D), lambda i: (0, 0)),
    out_shape=jax.ShapeDtypeStruct((G, 2 * D), jnp.float32),
)


def kernel(x, edge_index, batch, hls_attr, W0, b0, W1, b1, W2, b2,
           Wrel0, brel0, Wroot0, Wrel1, brel1, Wroot1, Wrel2, brel2, Wroot2):
    f32 = jnp.float32
    src = jnp.concatenate([edge_index[0], jnp.zeros((EP - E,), jnp.int32)])
    dst = jnp.concatenate([edge_index[1], jnp.full((EP - E,), N, jnp.int32)])
    xp = jnp.pad(x, ((0, NP - N), (0, 0)))
    batch_p = jnp.concatenate([batch, jnp.full((NP - N,), G, jnp.int32)])
    batch2 = batch_p.reshape(SR, SL)
    batch_row = batch_p.reshape(NP, 1)
    batch_lane = batch_p.reshape(1, NP)
    keep_i = (jnp.arange(NP, dtype=jnp.int32) < N).astype(jnp.int32)
    nmask2 = keep_i.astype(f32).reshape(SR, SL)
    nm_row = keep_i.astype(f32).reshape(NP, 1)

    params = [(W0, b0, Wrel0, brel0, Wroot0),
              (W1, b1, Wrel1, brel1, Wroot1),
              (W2, b2, Wrel2, brel2, Wroot2)]

    _sc_rows, _sc_scalar = _sc_kernels()
    degparts = _sc_scalar(jnp.ones((NP,), f32), keep_i, src, dst)
    x_cur = xp
    hs = []
    for l in range(3):
        W, b, Wrel, brel, Wroot = params[l]
        degt = degparts.T
        y, xw, dis = _tca(x_cur, W, degt, nm_row)
        accparts = _sc_rows(y, src, dst, keep_i)
        xnew = _tcb(accparts, xw, dis, nm_row, b.reshape(1, D))
        aggparts = _sc_rows(xnew, src, dst, keep_i)
        wrel8 = jnp.concatenate([Wrel, jnp.zeros((D, 7), f32)], axis=1)
        wroot8 = jnp.concatenate([Wroot, jnp.zeros((D, 7), f32)], axis=1)
        s4 = _tcb2(aggparts, xnew, wrel8, wroot8, brel.reshape(1, 1))
        s2 = s4[:, 0].reshape(SR, SL)
        keep2, tanh2 = _tcc1(s2, batch2, nmask2)
        keep_new_i = keep2.reshape(NP).astype(jnp.int32)
        xmod, h = _tcc2(xnew, tanh2.reshape(NP, 1), keep2.reshape(NP, 1),
                        batch_row, batch_lane)
        hs.append(h)
        if l < 2:
            degparts = _sc_scalar(keep2.reshape(NP), keep_new_i, src, dst)
        x_cur = xmod
        keep_i = keep_new_i
        nmask2 = keep2
        nm_row = keep2.reshape(NP, 1)

    hsum = _tcsum(hs[0], hs[1], hs[2])
    out = jnp.concatenate([hsum, hls_attr], axis=1)
    return (out, hs[0], hs[1], hs[2])


# speedup vs baseline: 1.1472x; 1.0071x over previous
import functools

import jax
import jax.numpy as jnp
import numpy as np
from jax import lax
from jax.experimental import pallas as pl
from jax.experimental.pallas import tpu as pltpu
from jax.experimental.pallas import tpu_sc as plsc

N = 10000
E = 320000
D = 128
G = 16

NP = 10240
NDUMP = N
EP = 327680
NC = 2
NS = 16
NW = NC * NS
CH = 128
EPT = EP // NW
NCHUNK = EPT // CH
RPT = NP // NS
QA = 120
QB = 2 * NCHUNK - QA
ROWB = 1280
NROWB = NP // ROWB
SR, SL = 8, 1280
INT_MIN = np.int32(-2147483648)
BIG = 3.0e38


def _sc_rows_body(y_hbm, src_hbm, dst_hbm, keep_hbm, out_hbm,
                  acc, keep_v, sbuf, draw, deff, rows, zrow, sem0, sem1):
    c = lax.axis_index("c")
    s = lax.axis_index("s")
    wid = s * NC + c
    for i in range(8):
        for j in range(D // 16):
            zrow[i, pl.ds(j * 16, 16)] = jnp.zeros((16,), jnp.float32)
    rbase = s * RPT

    @pl.loop(0, RPT // 8)
    def _zero(i):
        pltpu.sync_copy(zrow, acc.at[pl.ds(rbase + i * 8, 8)])

    pltpu.sync_copy(keep_hbm, keep_v)
    plsc.subcore_barrier()
    nch = jnp.where(c == 0, QA, QB)
    ebase = jnp.where(c == 0, s * (QA * CH), NS * QA * CH + s * (QB * CH))
    sems = (sem0, sem1)

    def load_and_prep(chunk, b):
        base = ebase + chunk * CH
        pltpu.sync_copy(src_hbm.at[pl.ds(base, CH)], sbuf.at[b])
        pltpu.sync_copy(dst_hbm.at[pl.ds(base, CH)], draw)
        for j in range(CH // 16):
            si = sbuf[b, pl.ds(j * 16, 16)]
            di = draw[pl.ds(j * 16, 16)]
            ks = plsc.load_gather(keep_v, [si])
            kd = plsc.load_gather(keep_v, [di])
            ok = jnp.logical_and(ks > 0, kd > 0)
            deff[b, pl.ds(j * 16, 16)] = jnp.where(ok, di, NDUMP)

    load_and_prep(0, 0)
    pltpu.async_copy(y_hbm.at[sbuf.at[0]], rows.at[0], sem0)

    @pl.loop(0, nch // 2)
    def _pair(i):
        for b in range(2):
            cur = 2 * i + b
            nb = 1 - b

            @pl.when(cur + 1 < nch)
            def _():
                load_and_prep(cur + 1, nb)
                pltpu.async_copy(y_hbm.at[sbuf.at[nb]], rows.at[nb], sems[nb])

            pltpu.make_async_copy(y_hbm.at[sbuf.at[b]], rows.at[b],
                                  sems[b]).wait()
            pltpu.sync_copy(rows.at[b], acc.at[deff.at[b]], add=True)

    plsc.subcore_barrier()
    pltpu.sync_copy(acc.at[pl.ds(rbase, RPT)], out_hbm.at[c, pl.ds(rbase, RPT)])


def _sc_scalar_body(vals_hbm, keep_hbm, src_hbm, dst_hbm, out_hbm,
                    acc1, vals_v, keep_v, sbuf, dbuf, vbuf, dbuf2, zb):
    c = lax.axis_index("c")
    s = lax.axis_index("s")
    wid = s * NC + c
    for j in range(8):
        zb[pl.ds(j * 16, 16)] = jnp.zeros((16,), jnp.float32)
    zbase = s * (NP // NS)

    @pl.loop(0, NP // NS // CH)
    def _zero(i):
        pltpu.sync_copy(zb, acc1.at[pl.ds(zbase + i * CH, CH)])

    pltpu.sync_copy(vals_hbm, vals_v)
    pltpu.sync_copy(keep_hbm, keep_v)
    plsc.subcore_barrier()
    ebase = wid * EPT

    @pl.loop(0, NCHUNK)
    def _chunk(g):
        base = ebase + g * CH
        pltpu.sync_copy(src_hbm.at[pl.ds(base, CH)], sbuf)
        pltpu.sync_copy(dst_hbm.at[pl.ds(base, CH)], dbuf)
        for j in range(CH // 16):
            si = sbuf[pl.ds(j * 16, 16)]
            di = dbuf[pl.ds(j * 16, 16)]
            ks = plsc.load_gather(keep_v, [si])
            kd = plsc.load_gather(keep_v, [di])
            pv = plsc.load_gather(vals_v, [si])
            val = jnp.where(jnp.logical_and(ks > 0, kd > 0), pv, 0.0)
            vbuf[0, pl.ds(j * 16, 16)] = val
            dbuf2[0, pl.ds(j * 16, 16)] = di
        pltpu.sync_copy(vbuf.at[0], acc1.at[dbuf2.at[0]], add=True)

    plsc.subcore_barrier()
    pltpu.sync_copy(acc1.at[pl.ds(zbase, NP // NS)],
                    out_hbm.at[c, pl.ds(zbase, NP // NS)])


@functools.cache
def _sc_kernels():
    mesh = plsc.VectorSubcoreMesh(core_axis_name="c", subcore_axis_name="s",
                                  num_cores=NC, num_subcores=NS)
    params = pltpu.CompilerParams(needs_layout_passes=False)
    sc_rows = pl.kernel(
        _sc_rows_body,
        out_type=jax.ShapeDtypeStruct((NC, NP, D), jnp.float32),
        mesh=mesh,
        compiler_params=params,
        scratch_types=[
            pltpu.VMEM_SHARED((NP, D), jnp.float32),
            pltpu.VMEM((NP,), jnp.int32),
            pltpu.VMEM((2, CH), jnp.int32),
            pltpu.VMEM((CH,), jnp.int32),
            pltpu.VMEM((2, CH), jnp.int32),
            pltpu.VMEM((2, CH, D), jnp.float32),
            pltpu.VMEM((8, D), jnp.float32),
            pltpu.SemaphoreType.DMA,
            pltpu.SemaphoreType.DMA,
        ],
    )
    sc_scalar = pl.kernel(
        _sc_scalar_body,
        out_type=jax.ShapeDtypeStruct((NC, NP), jnp.float32),
        mesh=mesh,
        compiler_params=params,
        scratch_types=[
            pltpu.VMEM_SHARED((NP,), jnp.float32),
            pltpu.VMEM((NP,), jnp.float32),
            pltpu.VMEM((NP,), jnp.int32),
            pltpu.VMEM((CH,), jnp.int32),
            pltpu.VMEM((CH,), jnp.int32),
            pltpu.VMEM((1, CH), jnp.float32),
            pltpu.VMEM((1, CH), jnp.int32),
            pltpu.VMEM((CH,), jnp.float32),
        ],
    )
    return sc_rows, sc_scalar


def _mmul(a, b):
    return jnp.dot(a, b, preferred_element_type=jnp.float32)


def _tca_body(x_ref, w_ref, degt_ref, nm_ref, y_ref, xw_ref, dis_ref):
    xw = _mmul(x_ref[...], w_ref[...])
    deg = jnp.sum(degt_ref[...], axis=1, keepdims=True) + nm_ref[...]
    dis = jnp.where(deg > 0, 1.0 / jnp.sqrt(jnp.maximum(deg, 1e-12)), 0.0)
    xw_ref[...] = xw
    y_ref[...] = xw * dis
    dis_ref[...] = dis


_tca = pl.pallas_call(
    _tca_body,
    grid=(NROWB,),
    in_specs=[
        pl.BlockSpec((ROWB, D), lambda i: (i, 0)),
        pl.BlockSpec((D, D), lambda i: (0, 0)),
        pl.BlockSpec((ROWB, NC), lambda i: (i, 0)),
        pl.BlockSpec((ROWB, 1), lambda i: (i, 0)),
    ],
    out_specs=[
        pl.BlockSpec((ROWB, D), lambda i: (i, 0)),
        pl.BlockSpec((ROWB, D), lambda i: (i, 0)),
        pl.BlockSpec((ROWB, 1), lambda i: (i, 0)),
    ],
    out_shape=[
        jax.ShapeDtypeStruct((NP, D), jnp.float32),
        jax.ShapeDtypeStruct((NP, D), jnp.float32),
        jax.ShapeDtypeStruct((NP, 1), jnp.float32),
    ],
)


def _tcb_body(accp_ref, xw_ref, dis_ref, nm_ref, b_ref, xnew_ref):
    acc = accp_ref[0] + accp_ref[1]
    dis = dis_ref[...]
    nm = nm_ref[...]
    gout = acc * dis + (dis * dis * nm) * xw_ref[...] + b_ref[...]
    xnew_ref[...] = jnp.maximum(gout * nm, 0.0)


_tcb = pl.pallas_call(
    _tcb_body,
    grid=(NROWB,),
    in_specs=[
        pl.BlockSpec((NC, ROWB, D), lambda i: (0, i, 0)),
        pl.BlockSpec((ROWB, D), lambda i: (i, 0)),
        pl.BlockSpec((ROWB, 1), lambda i: (i, 0)),
        pl.BlockSpec((ROWB, 1), lambda i: (i, 0)),
        pl.BlockSpec((1, D), lambda i: (0, 0)),
    ],
    out_specs=pl.BlockSpec((ROWB, D), lambda i: (i, 0)),
    out_shape=jax.ShapeDtypeStruct((NP, D), jnp.float32),
)


def _tcb2_body(aggp_ref, xnew_ref, wrel_ref, wroot_ref, brel_ref, s4_ref):
    agg = aggp_ref[0] + aggp_ref[1]
    s4_ref[...] = (_mmul(agg, wrel_ref[...]) + brel_ref[...]) \
        + _mmul(xnew_ref[...], wroot_ref[...])


_tcb2 = pl.pallas_call(
    _tcb2_body,
    grid=(NROWB,),
    in_specs=[
        pl.BlockSpec((NC, ROWB, D), lambda i: (0, i, 0)),
        pl.BlockSpec((ROWB, D), lambda i: (i, 0)),
        pl.BlockSpec((D, 8), lambda i: (0, 0)),
        pl.BlockSpec((D, 8), lambda i: (0, 0)),
        pl.BlockSpec((1, 1), lambda i: (0, 0)),
    ],
    out_specs=pl.BlockSpec((ROWB, 8), lambda i: (i, 0)),
    out_shape=jax.ShapeDtypeStruct((NP, 8), jnp.float32),
)


def _tcc1_body(s_ref, batch_ref, nm_ref, keep_ref, scale_ref):
    s = s_ref[...]
    nm = nm_ref[...]
    alive = nm > 0.0
    ib = lax.bitcast_convert_type(s, jnp.int32)
    key = jnp.where(ib >= 0, ib, ib ^ np.int32(0x7FFFFFFF))
    key = jnp.where(alive, key, INT_MIN)
    r_i = lax.broadcasted_iota(jnp.int32, (SR, SL), 0)
    c_i = lax.broadcasted_iota(jnp.int32, (SR, SL), 1)
    idxv = r_i * SL + c_i
    gi = lax.broadcasted_iota(jnp.int32, (G, SR, SL), 0)
    g3 = gi == batch_ref[...][None, :, :]
    af = jnp.logical_and(g3, alive[None])
    cnt = jnp.sum(af.astype(jnp.int32), axis=(1, 2))
    k = ((cnt + 1) // 2).reshape(G, 1, 1)
    lo = jnp.full((G, 1, 1), INT_MIN, jnp.int32)
    for b in range(31, -1, -1):
        cand = lo + (INT_MIN if b == 31 else np.int32(1 << b))
        ind = jnp.logical_and(af, key[None] >= cand)
        c2 = jnp.sum(ind.astype(jnp.int32), axis=(1, 2)).reshape(G, 1, 1)
        lo = jnp.where(c2 >= k, cand, lo)
    t = lo
    gtc = jnp.sum(jnp.logical_and(af, key[None] > t).astype(jnp.int32),
                  axis=(1, 2)).reshape(G, 1, 1)
    r = k - gtc
    eqi = jnp.logical_and(af, key[None] == t).astype(jnp.int32)
    u = jnp.zeros((G, 1, 1), jnp.int32)
    for b in range(14, -1, -1):
        cand = u + np.int32(1 << b)
        sc = jnp.sum(jnp.where(idxv[None] < cand, eqi, 0),
                     axis=(1, 2)).reshape(G, 1, 1)
        u = jnp.where(sc <= r, cand, u)
    tv = jnp.sum(jnp.where(g3, t, 0), axis=0)
    uv = jnp.sum(jnp.where(g3, u, 0), axis=0)
    keep = jnp.logical_and(
        alive,
        jnp.logical_or(key > tv,
                       jnp.logical_and(key == tv, idxv < uv)))
    keep_ref[...] = keep.astype(jnp.float32)
    scale_ref[...] = jnp.tanh(s)


_tcc1 = pl.pallas_call(
    _tcc1_body,
    grid=(1,),
    in_specs=[
        pl.BlockSpec((SR, SL), lambda i: (0, 0)),
        pl.BlockSpec((SR, SL), lambda i: (0, 0)),
        pl.BlockSpec((SR, SL), lambda i: (0, 0)),
    ],
    out_specs=[
        pl.BlockSpec((SR, SL), lambda i: (0, 0)),
        pl.BlockSpec((SR, SL), lambda i: (0, 0)),
    ],
    out_shape=[
        jax.ShapeDtypeStruct((SR, SL), jnp.float32),
        jax.ShapeDtypeStruct((SR, SL), jnp.float32),
    ],
)


def _tcc2_body(x_ref, sc_ref, kp_ref, br_ref, bl_ref, xm_ref, h_ref):
    i = pl.program_id(0)
    xm = (x_ref[...] * sc_ref[...]) * kp_ref[...]
    xm_ref[...] = xm

    @pl.when(i == 0)
    def _():
        h_ref[...] = jnp.concatenate(
            [jnp.full((G, D), -BIG, jnp.float32),
             jnp.zeros((G, D), jnp.float32)], axis=1)

    gm = (lax.broadcasted_iota(jnp.int32, (G, ROWB), 0)
          == bl_ref[...]).astype(jnp.float32)
    smp = jnp.dot(gm, xm, preferred_element_type=jnp.float32,
                  precision=lax.Precision.HIGHEST)
    h_ref[:, D:] += smp
    br = br_ref[...]
    kp = kp_ref[...]
    for g in range(G):
        m = jnp.logical_and(br == g, kp > 0.0)
        red = jnp.max(jnp.where(m, xm, -BIG), axis=0, keepdims=True)
        h_ref[g:g + 1, :D] = jnp.maximum(h_ref[g:g + 1, :D], red)

    @pl.when(i == NROWB - 1)
    def _():
        mx = h_ref[:, :D]
        h_ref[:, :D] = jnp.where(mx == -BIG, 0.0, mx)


_tcc2 = pl.pallas_call(
    _tcc2_body,
    grid=(NROWB,),
    in_specs=[
        pl.BlockSpec((ROWB, D), lambda i: (i, 0)),
        pl.BlockSpec((ROWB, 1), lambda i: (i, 0)),
        pl.BlockSpec((ROWB, 1), lambda i: (i, 0)),
        pl.BlockSpec((ROWB, 1), lambda i: (i, 0)),
        pl.BlockSpec((1, ROWB), lambda i: (0, i)),
    ],
    out_specs=[
        pl.BlockSpec((ROWB, D), lambda i: (i, 0)),
        pl.BlockSpec((G, 2 * D), lambda i: (0, 0)),
    ],
    out_shape=[
        jax.ShapeDtypeStruct((NP, D), jnp.float32),
        jax.ShapeDtypeStruct((G, 2 * D), jnp.float32),
    ],
)


def _tcsum_body(a_ref, b_ref, c_ref, o_ref):
    o_ref[...] = a_ref[...] + b_ref[...] + c_ref[...]


_tcsum = pl.pallas_call(
    _tcsum_body,
    grid=(1,),
    in_specs=[pl.BlockSpec((G, 2 * D), lambda i: (0, 0))] * 3,
    out_specs=pl.BlockSpec((G, 2 * D), lambda i: (0, 0)),
    out_shape=jax.ShapeDtypeStruct((G, 2 * D), jnp.float32),
)


def kernel(x, edge_index, batch, hls_attr, W0, b0, W1, b1, W2, b2,
           Wrel0, brel0, Wroot0, Wrel1, brel1, Wroot1, Wrel2, brel2, Wroot2):
    f32 = jnp.float32
    src = jnp.concatenate([edge_index[0], jnp.zeros((EP - E,), jnp.int32)])
    dst = jnp.concatenate([edge_index[1], jnp.full((EP - E,), N, jnp.int32)])
    xp = jnp.pad(x, ((0, NP - N), (0, 0)))
    batch_p = jnp.concatenate([batch, jnp.full((NP - N,), G, jnp.int32)])
    batch2 = batch_p.reshape(SR, SL)
    batch_row = batch_p.reshape(NP, 1)
    batch_lane = batch_p.reshape(1, NP)
    keep_i = (jnp.arange(NP, dtype=jnp.int32) < N).astype(jnp.int32)
    nmask2 = keep_i.astype(f32).reshape(SR, SL)
    nm_row = keep_i.astype(f32).reshape(NP, 1)

    params = [(W0, b0, Wrel0, brel0, Wroot0),
              (W1, b1, Wrel1, brel1, Wroot1),
              (W2, b2, Wrel2, brel2, Wroot2)]

    _sc_rows, _sc_scalar = _sc_kernels()
    degparts = _sc_scalar(jnp.ones((NP,), f32), keep_i, src, dst)
    x_cur = xp
    hs = []
    for l in range(3):
        W, b, Wrel, brel, Wroot = params[l]
        degt = degparts.T
        y, xw, dis = _tca(x_cur, W, degt, nm_row)
        accparts = _sc_rows(y, src, dst, keep_i)
        xnew = _tcb(accparts, xw, dis, nm_row, b.reshape(1, D))
        aggparts = _sc_rows(xnew, src, dst, keep_i)
        wrel8 = jnp.concatenate([Wrel, jnp.zeros((D, 7), f32)], axis=1)
        wroot8 = jnp.concatenate([Wroot, jnp.zeros((D, 7), f32)], axis=1)
        s4 = _tcb2(aggparts, xnew, wrel8, wroot8, brel.reshape(1, 1))
        s2 = s4[:, 0].reshape(SR, SL)
        keep2, tanh2 = _tcc1(s2, batch2, nmask2)
        keep_new_i = keep2.reshape(NP).astype(jnp.int32)
        xmod, h = _tcc2(xnew, tanh2.reshape(NP, 1), keep2.reshape(NP, 1),
                        batch_row, batch_lane)
        hs.append(h)
        if l < 2:
            degparts = _sc_scalar(keep2.reshape(NP), keep_new_i, src, dst)
        x_cur = xmod
        keep_i = keep_new_i
        nmask2 = keep2
        nm_row = keep2.reshape(NP, 1)

    hsum = _tcsum(hs[0], hs[1], hs[2])
    out = jnp.concatenate([hsum, hls_attr], axis=1)
    return (out, hs[0], hs[1], hs[2])
